# Initial kernel scaffold; baseline (speedup 1.0000x reference)
import jax
import jax.numpy as jnp
from jax.experimental import pallas as pl


def kernel(x, edge_index, Wl1, bl1, Wr1, Wl2, bl2, Wr2, W3, b3):
    raise NotImplementedError("write your pallas kernel here")


# speedup vs baseline: 4.2630x; 4.2630x over previous
import functools

import jax
import jax.numpy as jnp
from jax import lax
from jax.experimental import pallas as pl
from jax.experimental.pallas import tpu as pltpu
from jax.experimental.pallas import tpu_sc as plsc

N = 10000
E = 320000
D = 128
NC = 2
NS = 16
NW = NC * NS
CH = 128
CPW = 79
EPAD = NW * CH * CPW
NPAD = 10240
RPT = NPAD // NS


def _sc_agg_body(h_hbm, src_hbm, dst_hbm, zero_nd,
                 psum,
                 acc, sidx, didx, rows, gsem):
    c = lax.axis_index("c")
    s = lax.axis_index("s")
    wid = s * NC + c
    r0 = s * RPT

    pltpu.sync_copy(zero_nd.at[pl.ds(r0, RPT)], acc.at[pl.ds(r0, RPT)])
    plsc.subcore_barrier()

    @pl.loop(0, CPW)
    def _chunks(i):
        e0 = (wid + i * NW) * CH
        pltpu.sync_copy(src_hbm.at[pl.ds(e0, CH)], sidx)
        pltpu.sync_copy(dst_hbm.at[pl.ds(e0, CH)], didx.at[0])
        pltpu.async_copy(h_hbm.at[sidx], rows, gsem).wait()
        pltpu.sync_copy(rows, acc.at[didx.at[0]], add=True)

    plsc.subcore_barrier()

    o0 = c * NPAD + r0
    pltpu.sync_copy(acc.at[pl.ds(r0, RPT)], psum.at[pl.ds(o0, RPT)])


def _sc_cnt_body(dst_hbm, zero_nd, ones_nd,
                 cnt,
                 acc, didx, onesv):
    c = lax.axis_index("c")
    s = lax.axis_index("s")
    wid = s * NC + c
    r0 = s * RPT

    pltpu.sync_copy(zero_nd.at[pl.ds(r0, RPT)], acc.at[pl.ds(r0, RPT)])
    pltpu.sync_copy(ones_nd, onesv)
    plsc.subcore_barrier()

    @pl.loop(0, CPW)
    def _chunks(i):
        e0 = (wid + i * NW) * CH
        pltpu.sync_copy(dst_hbm.at[pl.ds(e0, CH)], didx.at[0])
        pltpu.sync_copy(onesv, acc.at[didx.at[0]], add=True)

    plsc.subcore_barrier()

    o0 = c * NPAD + r0
    pltpu.sync_copy(acc.at[pl.ds(r0, RPT)], cnt.at[pl.ds(o0, RPT)])


_sc_mesh = plsc.VectorSubcoreMesh(core_axis_name="c", subcore_axis_name="s",
                                  num_cores=NC, num_subcores=NS)

_sc_agg = pl.kernel(
    _sc_agg_body,
    out_type=jax.ShapeDtypeStruct((NC * NPAD, D), jnp.float32),
    mesh=_sc_mesh,
    scratch_types=(
        pltpu.VMEM_SHARED((NPAD, D), jnp.float32),
        pltpu.VMEM((CH,), jnp.int32),
        pltpu.VMEM((1, CH), jnp.int32),
        pltpu.VMEM((CH, D), jnp.float32),
        pltpu.SemaphoreType.DMA,
    ),
)

_sc_cnt = pl.kernel(
    _sc_cnt_body,
    out_type=jax.ShapeDtypeStruct((NC * NPAD, D), jnp.float32),
    mesh=_sc_mesh,
    scratch_types=(
        pltpu.VMEM_SHARED((NPAD, D), jnp.float32),
        pltpu.VMEM((1, CH), jnp.int32),
        pltpu.VMEM((CH, D), jnp.float32),
    ),
)


R = 1000


def _dense1_body(p0, p1, c0, c1, x, wl, wr, bl, out):
    tot = jnp.sum(c0[0] + c1[0], axis=1, keepdims=True) * (1.0 / D)
    inv = 1.0 / jnp.maximum(tot, 1.0)
    agg = (p0[0] + p1[0]) * inv
    h = lax.dot_general(agg, wl[...], (((1,), (1,)), ((), ())),
                        preferred_element_type=jnp.float32)
    h = h + bl[...] + lax.dot_general(x[...], wr[...], (((1,), (1,)), ((), ())),
                                      preferred_element_type=jnp.float32)
    out[...] = jnp.maximum(h, 0.0)


def _dense2_body(p0, p1, c0, c1, x, wl, wr, bl, w3, b3, out):
    tot = jnp.sum(c0[0] + c1[0], axis=1, keepdims=True) * (1.0 / D)
    inv = 1.0 / jnp.maximum(tot, 1.0)
    agg = (p0[0] + p1[0]) * inv
    h = lax.dot_general(agg, wl[...], (((1,), (1,)), ((), ())),
                        preferred_element_type=jnp.float32)
    h = h + bl[...] + lax.dot_general(x[...], wr[...], (((1,), (1,)), ((), ())),
                                      preferred_element_type=jnp.float32)
    h = jnp.maximum(h, 0.0)
    out[...] = lax.dot_general(h, w3[...], (((1,), (1,)), ((), ())),
                               preferred_element_type=jnp.float32) + b3[...]


_row_blk = pl.BlockSpec((R, D), lambda i: (i, 0))
_p0_blk = pl.BlockSpec((1, R, D), lambda i: (0, i, 0))
_p1_blk = pl.BlockSpec((1, R, D), lambda i: (1, i, 0))
_w_blk = pl.BlockSpec((D, D), lambda i: (0, 0))
_b_blk = pl.BlockSpec((1, D), lambda i: (0, 0))

_dense1 = pl.pallas_call(
    _dense1_body,
    grid=(N // R,),
    in_specs=[_p0_blk, _p1_blk, _p0_blk, _p1_blk, _row_blk,
              _w_blk, _w_blk, _b_blk],
    out_specs=_row_blk,
    out_shape=jax.ShapeDtypeStruct((N, D), jnp.float32),
)

_dense2 = pl.pallas_call(
    _dense2_body,
    grid=(N // R,),
    in_specs=[_p0_blk, _p1_blk, _p0_blk, _p1_blk, _row_blk,
              _w_blk, _w_blk, _b_blk, _w_blk, _b_blk],
    out_specs=_row_blk,
    out_shape=jax.ShapeDtypeStruct((N, D), jnp.float32),
)


def kernel(x, edge_index, Wl1, bl1, Wr1, Wl2, bl2, Wr2, W3, b3):
    src = edge_index[0]
    dst = edge_index[1]
    pad = EPAD - E
    srcp = jnp.concatenate([src, jnp.zeros((pad,), src.dtype)])
    dstp = jnp.concatenate([dst, jnp.full((pad,), N, dst.dtype)])
    zero_nd = jnp.zeros((NPAD, D), jnp.float32)
    ones_nd = jnp.ones((CH, D), jnp.float32)
    bl1r = bl1.reshape(1, D)
    bl2r = bl2.reshape(1, D)
    b3r = b3.reshape(1, D)

    cnt = _sc_cnt(dstp, zero_nd, ones_nd)
    cn = cnt.reshape(NC, NPAD, D)
    psum1 = _sc_agg(x, srcp, dstp, zero_nd)
    p1 = psum1.reshape(NC, NPAD, D)
    h = _dense1(p1, p1, cn, cn, x, Wl1, Wr1, bl1r)

    psum2 = _sc_agg(h, srcp, dstp, zero_nd)
    p2 = psum2.reshape(NC, NPAD, D)
    return _dense2(p2, p2, cn, cn, h, Wl2, Wr2, bl2r, W3, b3r)
